# Initial kernel scaffold; baseline (speedup 1.0000x reference)
import jax
import jax.numpy as jnp
from jax.experimental import pallas as pl


def kernel(x, edge_index, W1, b1, W2, b2):
    raise NotImplementedError("write your pallas kernel here")


# speedup vs baseline: 28.7672x; 28.7672x over previous
import functools

import jax
import jax.numpy as jnp
from jax import lax
from jax.experimental import pallas as pl
from jax.experimental.pallas import tpu as pltpu
from jax.experimental.pallas import tpu_sc as plsc

N = 10000
E = 320000
D_IN, D_H, D_OUT = 128, 16, 7

NC, NS = 2, 16
NW = NC * NS
EPT = E // NW
SUB = 80
NSUB = EPT // SUB
RPS = N // NS

ROWB = 2000


@functools.cache
def _sc_mesh():
    return plsc.VectorSubcoreMesh(
        core_axis_name="c", subcore_axis_name="s", num_cores=NC, num_subcores=NS
    )


@functools.cache
def _sc_deg():
    @functools.partial(
        pl.kernel,
        out_type=jax.ShapeDtypeStruct((NC, N, D_H), jnp.float32),
        mesh=_sc_mesh(),
        scratch_types=[
            pltpu.VMEM((NSUB, SUB), jnp.int32),
            pltpu.VMEM((SUB, D_H), jnp.float32),
            pltpu.VMEM_SHARED((N, D_H), jnp.float32),
        ],
        compiler_params=pltpu.CompilerParams(use_tc_tiling_on_sc=False),
    )
    def deg_kernel(dst_hbm, ones_hbm, zrows_hbm, out_hbm, idx_d, ones_v, acc_sh):
        c = lax.axis_index("c")
        s = lax.axis_index("s")
        tile = c * NS + s
        pltpu.sync_copy(zrows_hbm, acc_sh.at[pl.ds(s * RPS, RPS)])
        pltpu.sync_copy(dst_hbm.at[tile], idx_d)
        pltpu.sync_copy(ones_hbm, ones_v)
        plsc.subcore_barrier()

        def body(j, carry):
            pltpu.sync_copy(ones_v, acc_sh.at[idx_d.at[j]], add=True)
            return carry

        lax.fori_loop(0, NSUB, body, 0)
        plsc.subcore_barrier()
        pltpu.sync_copy(
            acc_sh.at[pl.ds(s * RPS, RPS)], out_hbm.at[c, pl.ds(s * RPS, RPS)]
        )

    return deg_kernel


@functools.cache
def _sc_agg():
    @functools.partial(
        pl.kernel,
        out_type=jax.ShapeDtypeStruct((NC, N, D_H), jnp.float32),
        mesh=_sc_mesh(),
        scratch_types=[
            pltpu.VMEM((NSUB, SUB), jnp.int32),
            pltpu.VMEM((NSUB, SUB), jnp.int32),
            pltpu.VMEM((SUB, D_H), jnp.float32),
            pltpu.VMEM_SHARED((N, D_H), jnp.float32),
            pltpu.SemaphoreType.DMA,
        ],
        compiler_params=pltpu.CompilerParams(use_tc_tiling_on_sc=False),
    )
    def agg_kernel(g_hbm, src_hbm, dst_hbm, zrows_hbm, out_hbm,
                   idx_s, idx_d, rows, acc_sh, sem):
        c = lax.axis_index("c")
        s = lax.axis_index("s")
        tile = c * NS + s
        pltpu.sync_copy(zrows_hbm, acc_sh.at[pl.ds(s * RPS, RPS)])
        pltpu.sync_copy(src_hbm.at[tile], idx_s)
        pltpu.sync_copy(dst_hbm.at[tile], idx_d)
        plsc.subcore_barrier()

        def body(j, carry):
            pltpu.async_copy(g_hbm.at[idx_s.at[j]], rows, sem).wait()
            pltpu.sync_copy(rows, acc_sh.at[idx_d.at[j]], add=True)
            return carry

        lax.fori_loop(0, NSUB, body, 0)
        plsc.subcore_barrier()
        pltpu.sync_copy(
            acc_sh.at[pl.ds(s * RPS, RPS)], out_hbm.at[c, pl.ds(s * RPS, RPS)]
        )

    return agg_kernel


def _k2_body(degp_ref, x_ref, w1_ref, g1_ref, dinv_ref):
    deg = degp_ref[0] + degp_ref[1] + 1.0
    dinv = lax.rsqrt(deg)
    h = jnp.dot(x_ref[...], w1_ref[...], preferred_element_type=jnp.float32)
    g1_ref[...] = h * dinv
    dinv_ref[...] = dinv


def _tc_k2(degp, x, w1):
    grid = (N // ROWB,)
    return pl.pallas_call(
        _k2_body,
        grid=grid,
        in_specs=[
            pl.BlockSpec((NC, ROWB, D_H), lambda i: (0, i, 0)),
            pl.BlockSpec((ROWB, D_IN), lambda i: (i, 0)),
            pl.BlockSpec((D_IN, D_H), lambda i: (0, 0)),
        ],
        out_specs=[
            pl.BlockSpec((ROWB, D_H), lambda i: (i, 0)),
            pl.BlockSpec((ROWB, D_H), lambda i: (i, 0)),
        ],
        out_shape=[
            jax.ShapeDtypeStruct((N, D_H), jnp.float32),
            jax.ShapeDtypeStruct((N, D_H), jnp.float32),
        ],
    )(degp, x, w1)


def _k4_body(accp_ref, g1_ref, dinv_ref, b1_ref, w2_ref, g2_ref):
    acc = accp_ref[0] + accp_ref[1] + g1_ref[...]
    h1 = jnp.maximum(acc * dinv_ref[...] + b1_ref[...], 0.0)
    g2 = jnp.dot(h1, w2_ref[...], preferred_element_type=jnp.float32)
    g2_ref[...] = g2 * dinv_ref[...]


def _tc_k4(accp, g1, dinv, b1, w2p):
    grid = (N // ROWB,)
    return pl.pallas_call(
        _k4_body,
        grid=grid,
        in_specs=[
            pl.BlockSpec((NC, ROWB, D_H), lambda i: (0, i, 0)),
            pl.BlockSpec((ROWB, D_H), lambda i: (i, 0)),
            pl.BlockSpec((ROWB, D_H), lambda i: (i, 0)),
            pl.BlockSpec((1, D_H), lambda i: (0, 0)),
            pl.BlockSpec((D_H, D_H), lambda i: (0, 0)),
        ],
        out_specs=pl.BlockSpec((ROWB, D_H), lambda i: (i, 0)),
        out_shape=jax.ShapeDtypeStruct((N, D_H), jnp.float32),
    )(accp, g1, dinv, b1, w2p)


def _k6_body(accp_ref, g2_ref, dinv_ref, b2_ref, out_ref):
    o = (accp_ref[0] + accp_ref[1] + g2_ref[...]) * dinv_ref[...] + b2_ref[...]
    valid = lax.broadcasted_iota(jnp.int32, o.shape, 1) < D_OUT
    o = jnp.where(valid, o, -1e30)
    m = jnp.max(o, axis=1, keepdims=True)
    ex = jnp.exp(o - m)
    lse = jnp.log(jnp.sum(ex, axis=1, keepdims=True))
    out_ref[...] = o - m - lse


def _tc_k6(accp, g2, dinv, b2p):
    grid = (N // ROWB,)
    return pl.pallas_call(
        _k6_body,
        grid=grid,
        in_specs=[
            pl.BlockSpec((NC, ROWB, D_H), lambda i: (0, i, 0)),
            pl.BlockSpec((ROWB, D_H), lambda i: (i, 0)),
            pl.BlockSpec((ROWB, D_H), lambda i: (i, 0)),
            pl.BlockSpec((1, D_H), lambda i: (0, 0)),
        ],
        out_specs=pl.BlockSpec((ROWB, D_H), lambda i: (i, 0)),
        out_shape=jax.ShapeDtypeStruct((N, D_H), jnp.float32),
    )(accp, g2, dinv, b2p)


def kernel(x, edge_index, W1, b1, W2, b2):
    src = edge_index[0].reshape(NW, NSUB, SUB)
    dst = edge_index[1].reshape(NW, NSUB, SUB)
    zrows = jnp.zeros((RPS, D_H), jnp.float32)
    ones = jnp.ones((SUB, D_H), jnp.float32)

    degp = _sc_deg()(dst, ones, zrows)
    g1, dinv = _tc_k2(degp, x, W1)
    accp1 = _sc_agg()(g1, src, dst, zrows)

    w2p = jnp.zeros((D_H, D_H), jnp.float32).at[:, :D_OUT].set(W2)
    g2 = _tc_k4(accp1, g1, dinv, b1.reshape(1, D_H), w2p)
    accp2 = _sc_agg()(g2, src, dst, zrows)

    b2p = jnp.zeros((1, D_H), jnp.float32).at[0, :D_OUT].set(b2)
    out16 = _tc_k6(accp2, g2, dinv, b2p)
    return out16[:, :D_OUT]


# speedup vs baseline: 53.8727x; 1.8727x over previous
import functools

import jax
import jax.numpy as jnp
from jax import lax
from jax.experimental import pallas as pl
from jax.experimental.pallas import tpu as pltpu
from jax.experimental.pallas import tpu_sc as plsc

N = 10000
E = 320000
D_IN, D_H, D_OUT = 128, 16, 7

NC, NS = 2, 16
NW = NC * NS
EPT = E // NW
SUB = 80
NSUB = EPT // SUB
RPS = N // NS
NBUF = 5

ROWB = 2000


@functools.cache
def _sc_mesh():
    return plsc.VectorSubcoreMesh(
        core_axis_name="c", subcore_axis_name="s", num_cores=NC, num_subcores=NS
    )


@functools.cache
def _sc_deg():
    @functools.partial(
        pl.kernel,
        out_type=jax.ShapeDtypeStruct((NC, N), jnp.float32),
        mesh=_sc_mesh(),
        scratch_types=[
            pltpu.VMEM((NSUB, SUB), jnp.int32),
            pltpu.VMEM((SUB,), jnp.float32),
            pltpu.VMEM_SHARED((N,), jnp.float32),
            pltpu.SemaphoreType.DMA,
        ],
        compiler_params=pltpu.CompilerParams(use_tc_tiling_on_sc=False),
    )
    def deg_kernel(dst_hbm, ones_hbm, zeros_hbm, out_hbm, idx_d, ones_v, acc_sh, sem):
        c = lax.axis_index("c")
        s = lax.axis_index("s")
        tile = c * NS + s

        @pl.when(s == 0)
        def _zero():
            pltpu.sync_copy(zeros_hbm, acc_sh)

        pltpu.sync_copy(dst_hbm.at[tile], idx_d)
        pltpu.sync_copy(ones_hbm, ones_v)
        plsc.subcore_barrier()

        W = 8
        for b in range(W):
            pltpu.async_copy(ones_v, acc_sh.at[idx_d.at[b]], sem, add=True)

        @pl.loop(W, NSUB)
        def _step(j):
            pltpu.make_async_copy(ones_v, acc_sh.at[pl.ds(0, SUB)], sem).wait()
            pltpu.async_copy(ones_v, acc_sh.at[idx_d.at[j]], sem, add=True)

        for b in range(W):
            pltpu.make_async_copy(ones_v, acc_sh.at[pl.ds(0, SUB)], sem).wait()
        plsc.subcore_barrier()

        @pl.when(s == 0)
        def _writeback():
            pltpu.sync_copy(acc_sh, out_hbm.at[c])

    return deg_kernel


@functools.cache
def _sc_agg(width):
    @functools.partial(
        pl.kernel,
        out_type=jax.ShapeDtypeStruct((NC, N, width), jnp.float32),
        mesh=_sc_mesh(),
        scratch_types=[
            pltpu.VMEM((NSUB, SUB), jnp.int32),
            pltpu.VMEM((NSUB, SUB), jnp.int32),
            pltpu.VMEM((NBUF, SUB, width), jnp.float32),
            pltpu.VMEM_SHARED((N, width), jnp.float32),
            pltpu.SemaphoreType.DMA((NBUF,)),
            pltpu.SemaphoreType.DMA((NBUF,)),
        ],
        compiler_params=pltpu.CompilerParams(use_tc_tiling_on_sc=False),
    )
    def agg_kernel(g_hbm, src_hbm, dst_hbm, zrows_hbm, out_hbm,
                   idx_s, idx_d, rows, acc_sh, gsem, ssem):
        c = lax.axis_index("c")
        s = lax.axis_index("s")
        tile = c * NS + s
        pltpu.sync_copy(zrows_hbm, acc_sh.at[pl.ds(s * RPS, RPS)])
        pltpu.sync_copy(src_hbm.at[tile], idx_s)
        pltpu.sync_copy(dst_hbm.at[tile], idx_d)
        plsc.subcore_barrier()

        for b in range(NBUF):
            pltpu.async_copy(g_hbm.at[idx_s.at[b]], rows.at[b], gsem.at[b])

        @pl.loop(0, NSUB, step=NBUF)
        def _group(j0):
            for b in range(NBUF):
                pltpu.make_async_copy(
                    g_hbm.at[idx_s.at[j0 + b]], rows.at[b], gsem.at[b]
                ).wait()
                pltpu.async_copy(
                    rows.at[b], acc_sh.at[idx_d.at[j0 + b]], ssem.at[b], add=True
                )
            for b in range(NBUF):
                jn = j0 + NBUF + b
                pltpu.make_async_copy(
                    rows.at[b], acc_sh.at[pl.ds(0, SUB)], ssem.at[b]
                ).wait()

                @pl.when(jn < NSUB)
                def _prefetch():
                    pltpu.async_copy(g_hbm.at[idx_s.at[jn]], rows.at[b], gsem.at[b])

        plsc.subcore_barrier()
        pltpu.sync_copy(
            acc_sh.at[pl.ds(s * RPS, RPS)], out_hbm.at[c, pl.ds(s * RPS, RPS)]
        )

    return agg_kernel


def _k2_body(degp_ref, x_ref, w1_ref, g1_ref, dinv_ref):
    deg = degp_ref[0] + degp_ref[1] + 1.0
    dinv = jnp.broadcast_to(lax.rsqrt(deg), (ROWB, D_H))
    h = jnp.dot(x_ref[...], w1_ref[...], preferred_element_type=jnp.float32)
    g1_ref[...] = h * dinv
    dinv_ref[...] = dinv


def _tc_k2(degp, x, w1):
    grid = (N // ROWB,)
    return pl.pallas_call(
        _k2_body,
        grid=grid,
        in_specs=[
            pl.BlockSpec((NC, ROWB, 1), lambda i: (0, i, 0)),
            pl.BlockSpec((ROWB, D_IN), lambda i: (i, 0)),
            pl.BlockSpec((D_IN, D_H), lambda i: (0, 0)),
        ],
        out_specs=[
            pl.BlockSpec((ROWB, D_H), lambda i: (i, 0)),
            pl.BlockSpec((ROWB, D_H), lambda i: (i, 0)),
        ],
        out_shape=[
            jax.ShapeDtypeStruct((N, D_H), jnp.float32),
            jax.ShapeDtypeStruct((N, D_H), jnp.float32),
        ],
    )(degp, x, w1)


def _k4_body(accp_ref, g1_ref, dinv_ref, b1_ref, w2_ref, g2_ref):
    acc = accp_ref[0] + accp_ref[1] + g1_ref[...]
    h1 = jnp.maximum(acc * dinv_ref[...] + b1_ref[...], 0.0)
    g2 = jnp.dot(h1, w2_ref[...], preferred_element_type=jnp.float32)
    g2_ref[...] = g2 * dinv_ref[...]


def _tc_k4(accp, g1, dinv, b1, w2p):
    grid = (N // ROWB,)
    return pl.pallas_call(
        _k4_body,
        grid=grid,
        in_specs=[
            pl.BlockSpec((NC, ROWB, D_H), lambda i: (0, i, 0)),
            pl.BlockSpec((ROWB, D_H), lambda i: (i, 0)),
            pl.BlockSpec((ROWB, D_H), lambda i: (i, 0)),
            pl.BlockSpec((1, D_H), lambda i: (0, 0)),
            pl.BlockSpec((D_H, D_H), lambda i: (0, 0)),
        ],
        out_specs=pl.BlockSpec((ROWB, D_H), lambda i: (i, 0)),
        out_shape=jax.ShapeDtypeStruct((N, D_H), jnp.float32),
    )(accp, g1, dinv, b1, w2p)


def _k6_body(accp_ref, g2_ref, dinv_ref, b2_ref, out_ref):
    o = (accp_ref[0] + accp_ref[1] + g2_ref[...]) * dinv_ref[...] + b2_ref[...]
    valid = lax.broadcasted_iota(jnp.int32, o.shape, 1) < D_OUT
    o = jnp.where(valid, o, -1e30)
    m = jnp.max(o, axis=1, keepdims=True)
    ex = jnp.exp(o - m)
    lse = jnp.log(jnp.sum(ex, axis=1, keepdims=True))
    out_ref[...] = o - m - lse


def _tc_k6(accp, g2, dinv, b2p):
    grid = (N // ROWB,)
    return pl.pallas_call(
        _k6_body,
        grid=grid,
        in_specs=[
            pl.BlockSpec((NC, ROWB, D_H), lambda i: (0, i, 0)),
            pl.BlockSpec((ROWB, D_H), lambda i: (i, 0)),
            pl.BlockSpec((ROWB, D_H), lambda i: (i, 0)),
            pl.BlockSpec((1, D_H), lambda i: (0, 0)),
        ],
        out_specs=pl.BlockSpec((ROWB, D_H), lambda i: (i, 0)),
        out_shape=jax.ShapeDtypeStruct((N, D_H), jnp.float32),
    )(accp, g2, dinv, b2p)


def kernel(x, edge_index, W1, b1, W2, b2):
    src = edge_index[0].reshape(NW, NSUB, SUB)
    dst = edge_index[1].reshape(NW, NSUB, SUB)
    zrows = jnp.zeros((RPS, D_H), jnp.float32)
    ones = jnp.ones((SUB,), jnp.float32)
    zeros_n = jnp.zeros((N,), jnp.float32)

    degp = _sc_deg()(dst, ones, zeros_n).reshape(NC, N, 1)
    g1, dinv = _tc_k2(degp, x, W1)
    accp1 = _sc_agg(D_H)(g1, src, dst, zrows)

    w2p = jnp.zeros((D_H, D_H), jnp.float32).at[:, :D_OUT].set(W2)
    g2 = _tc_k4(accp1, g1, dinv, b1.reshape(1, D_H), w2p)
    accp2 = _sc_agg(D_H)(g2, src, dst, zrows)

    b2p = jnp.zeros((1, D_H), jnp.float32).at[0, :D_OUT].set(b2)
    out16 = _tc_k6(accp2, g2, dinv, b2p)
    return out16[:, :D_OUT]


# speedup vs baseline: 62.7320x; 1.1644x over previous
import functools

import jax
import jax.numpy as jnp
from jax import lax
from jax.experimental import pallas as pl
from jax.experimental.pallas import tpu as pltpu
from jax.experimental.pallas import tpu_sc as plsc

N = 10000
E = 320000
D_IN, D_H, D_OUT = 128, 16, 7
D2 = 8

NC, NS = 2, 16
NW = NC * NS
CW = 128
NCHUNK = E // CW
BASE = NCHUNK // NW
XTRA = NCHUNK % NW
RPS = N // NS
NBUF = 6

ROWB = 2000


@functools.cache
def _sc_mesh():
    return plsc.VectorSubcoreMesh(
        core_axis_name="c", subcore_axis_name="s", num_cores=NC, num_subcores=NS
    )


def _stage_idx(src_hbm, idx_ref, start, tile):
    @pl.when(tile < XTRA)
    def _full():
        pltpu.sync_copy(src_hbm.at[pl.ds(start, BASE + 1)], idx_ref)

    @pl.when(tile >= XTRA)
    def _base():
        pltpu.sync_copy(
            src_hbm.at[pl.ds(start, BASE)], idx_ref.at[pl.ds(0, BASE)]
        )


@functools.cache
def _sc_deg():
    @functools.partial(
        pl.kernel,
        out_type=jax.ShapeDtypeStruct((NC, N), jnp.float32),
        mesh=_sc_mesh(),
        scratch_types=[
            pltpu.VMEM((BASE + 1, CW), jnp.int32),
            pltpu.VMEM((CW,), jnp.float32),
            pltpu.VMEM_SHARED((N,), jnp.float32),
            pltpu.SemaphoreType.DMA,
        ],
        compiler_params=pltpu.CompilerParams(use_tc_tiling_on_sc=False),
    )
    def deg_kernel(dst_hbm, ones_hbm, zeros_hbm, out_hbm, idx_d, ones_v, acc_sh, sem):
        c = lax.axis_index("c")
        s = lax.axis_index("s")
        tile = c * NS + s

        @pl.when(s == 0)
        def _zero():
            pltpu.sync_copy(zeros_hbm, acc_sh)

        start = BASE * tile + jnp.minimum(tile, XTRA)
        _stage_idx(dst_hbm, idx_d, start, tile)
        pltpu.sync_copy(ones_hbm, ones_v)
        plsc.subcore_barrier()

        W = 8
        for b in range(W):
            pltpu.async_copy(ones_v, acc_sh.at[idx_d.at[b]], sem, add=True)

        @pl.loop(W, BASE)
        def _step(j):
            pltpu.make_async_copy(ones_v, acc_sh.at[pl.ds(0, CW)], sem).wait()
            pltpu.async_copy(ones_v, acc_sh.at[idx_d.at[j]], sem, add=True)

        @pl.when(tile < XTRA)
        def _extra():
            pltpu.make_async_copy(ones_v, acc_sh.at[pl.ds(0, CW)], sem).wait()
            pltpu.async_copy(ones_v, acc_sh.at[idx_d.at[BASE]], sem, add=True)

        for b in range(W):
            pltpu.make_async_copy(ones_v, acc_sh.at[pl.ds(0, CW)], sem).wait()
        plsc.subcore_barrier()

        @pl.when(s == 0)
        def _writeback():
            pltpu.sync_copy(acc_sh, out_hbm.at[c])

    return deg_kernel


@functools.cache
def _sc_agg(width):
    @functools.partial(
        pl.kernel,
        out_type=jax.ShapeDtypeStruct((NC, N, width), jnp.float32),
        mesh=_sc_mesh(),
        scratch_types=[
            pltpu.VMEM((BASE + 1, CW), jnp.int32),
            pltpu.VMEM((BASE + 1, CW), jnp.int32),
            pltpu.VMEM((NBUF, CW, width), jnp.float32),
            pltpu.VMEM_SHARED((N, width), jnp.float32),
            pltpu.SemaphoreType.DMA((NBUF,)),
            pltpu.SemaphoreType.DMA((NBUF,)),
        ],
        compiler_params=pltpu.CompilerParams(use_tc_tiling_on_sc=False),
    )
    def agg_kernel(g_hbm, src_hbm, dst_hbm, zrows_hbm, out_hbm,
                   idx_s, idx_d, rows, acc_sh, gsem, ssem):
        c = lax.axis_index("c")
        s = lax.axis_index("s")
        tile = c * NS + s
        pltpu.sync_copy(zrows_hbm, acc_sh.at[pl.ds(s * RPS, RPS)])
        start = BASE * tile + jnp.minimum(tile, XTRA)
        _stage_idx(src_hbm, idx_s, start, tile)
        _stage_idx(dst_hbm, idx_d, start, tile)
        plsc.subcore_barrier()

        for b in range(NBUF):
            pltpu.async_copy(g_hbm.at[idx_s.at[b]], rows.at[b], gsem.at[b])

        @pl.loop(0, BASE, step=NBUF)
        def _group(j0):
            for b in range(NBUF):
                pltpu.make_async_copy(
                    g_hbm.at[idx_s.at[j0 + b]], rows.at[b], gsem.at[b]
                ).wait()
                pltpu.async_copy(
                    rows.at[b], acc_sh.at[idx_d.at[j0 + b]], ssem.at[b], add=True
                )
            for b in range(NBUF):
                jn = j0 + NBUF + b
                pltpu.make_async_copy(
                    rows.at[b], acc_sh.at[pl.ds(0, CW)], ssem.at[b]
                ).wait()

                @pl.when(jn < BASE + jnp.where(tile < XTRA, 1, 0))
                def _prefetch():
                    pltpu.async_copy(g_hbm.at[idx_s.at[jn]], rows.at[b], gsem.at[b])

        bt = BASE % NBUF

        @pl.when(tile < XTRA)
        def _tail():
            pltpu.make_async_copy(
                g_hbm.at[idx_s.at[BASE]], rows.at[bt], gsem.at[bt]
            ).wait()
            pltpu.async_copy(
                rows.at[bt], acc_sh.at[idx_d.at[BASE]], ssem.at[bt], add=True
            )
            pltpu.make_async_copy(
                rows.at[bt], acc_sh.at[pl.ds(0, CW)], ssem.at[bt]
            ).wait()

        plsc.subcore_barrier()
        pltpu.sync_copy(
            acc_sh.at[pl.ds(s * RPS, RPS)], out_hbm.at[c, pl.ds(s * RPS, RPS)]
        )

    return agg_kernel


def _k2_body(degp_ref, x_ref, w1_ref, g1_ref, dinv_ref):
    deg = degp_ref[0] + degp_ref[1] + 1.0
    dinv = jnp.broadcast_to(lax.rsqrt(deg), (ROWB, D_H))
    h = jnp.dot(x_ref[...], w1_ref[...], preferred_element_type=jnp.float32)
    g1_ref[...] = h * dinv
    dinv_ref[...] = dinv


def _tc_k2(degp, x, w1):
    grid = (N // ROWB,)
    return pl.pallas_call(
        _k2_body,
        grid=grid,
        in_specs=[
            pl.BlockSpec((NC, ROWB, 1), lambda i: (0, i, 0)),
            pl.BlockSpec((ROWB, D_IN), lambda i: (i, 0)),
            pl.BlockSpec((D_IN, D_H), lambda i: (0, 0)),
        ],
        out_specs=[
            pl.BlockSpec((ROWB, D_H), lambda i: (i, 0)),
            pl.BlockSpec((ROWB, D_H), lambda i: (i, 0)),
        ],
        out_shape=[
            jax.ShapeDtypeStruct((N, D_H), jnp.float32),
            jax.ShapeDtypeStruct((N, D_H), jnp.float32),
        ],
    )(degp, x, w1)


def _k4_body(accp_ref, g1_ref, dinv_ref, b1_ref, w2_ref, g2_ref):
    acc = accp_ref[0] + accp_ref[1] + g1_ref[...]
    h1 = jnp.maximum(acc * dinv_ref[...] + b1_ref[...], 0.0)
    g2 = jnp.dot(h1, w2_ref[...], preferred_element_type=jnp.float32)
    g2_ref[...] = g2 * dinv_ref[:, :D2]


def _tc_k4(accp, g1, dinv, b1, w2p):
    grid = (N // ROWB,)
    return pl.pallas_call(
        _k4_body,
        grid=grid,
        in_specs=[
            pl.BlockSpec((NC, ROWB, D_H), lambda i: (0, i, 0)),
            pl.BlockSpec((ROWB, D_H), lambda i: (i, 0)),
            pl.BlockSpec((ROWB, D_H), lambda i: (i, 0)),
            pl.BlockSpec((1, D_H), lambda i: (0, 0)),
            pl.BlockSpec((D_H, D2), lambda i: (0, 0)),
        ],
        out_specs=pl.BlockSpec((ROWB, D2), lambda i: (i, 0)),
        out_shape=jax.ShapeDtypeStruct((N, D2), jnp.float32),
    )(accp, g1, dinv, b1, w2p)


def _k6_body(accp_ref, g2_ref, dinv_ref, b2_ref, out_ref):
    o = (accp_ref[0] + accp_ref[1] + g2_ref[...]) * dinv_ref[:, :D2] + b2_ref[...]
    valid = lax.broadcasted_iota(jnp.int32, o.shape, 1) < D_OUT
    o = jnp.where(valid, o, -1e30)
    m = jnp.max(o, axis=1, keepdims=True)
    ex = jnp.exp(o - m)
    lse = jnp.log(jnp.sum(ex, axis=1, keepdims=True))
    out_ref[...] = o - m - lse


def _tc_k6(accp, g2, dinv, b2p):
    grid = (N // ROWB,)
    return pl.pallas_call(
        _k6_body,
        grid=grid,
        in_specs=[
            pl.BlockSpec((NC, ROWB, D2), lambda i: (0, i, 0)),
            pl.BlockSpec((ROWB, D2), lambda i: (i, 0)),
            pl.BlockSpec((ROWB, D_H), lambda i: (i, 0)),
            pl.BlockSpec((1, D2), lambda i: (0, 0)),
        ],
        out_specs=pl.BlockSpec((ROWB, D2), lambda i: (i, 0)),
        out_shape=jax.ShapeDtypeStruct((N, D2), jnp.float32),
    )(accp, g2, dinv, b2p)


def kernel(x, edge_index, W1, b1, W2, b2):
    src = edge_index[0].reshape(NCHUNK, CW)
    dst = edge_index[1].reshape(NCHUNK, CW)
    zrows = jnp.zeros((RPS, D_H), jnp.float32)
    ones = jnp.ones((CW,), jnp.float32)
    zeros_n = jnp.zeros((N,), jnp.float32)

    degp = _sc_deg()(dst, ones, zeros_n).reshape(NC, N, 1)
    g1, dinv = _tc_k2(degp, x, W1)
    accp1 = _sc_agg(D_H)(g1, src, dst, zrows)

    w2p = jnp.zeros((D_H, D2), jnp.float32).at[:, :D_OUT].set(W2)
    g2 = _tc_k4(accp1, g1, dinv, b1.reshape(1, D_H), w2p)
    accp2 = _sc_agg(D2)(g2, src, dst, jnp.zeros((RPS, D2), jnp.float32))

    b2p = jnp.zeros((1, D2), jnp.float32).at[0, :D_OUT].set(b2)
    out16 = _tc_k6(accp2, g2, dinv, b2p)
    return out16[:, :D_OUT]


# speedup vs baseline: 92.9890x; 1.4823x over previous
import functools

import jax
import jax.numpy as jnp
from jax import lax
from jax.experimental import pallas as pl
from jax.experimental.pallas import tpu as pltpu
from jax.experimental.pallas import tpu_sc as plsc

N = 10000
E = 320000
D_IN, D_H, D_OUT = 128, 16, 7
D2 = 8

NC, NS = 2, 16
NW = NC * NS
CW = 128
NCHUNK = E // CW
BASE = NCHUNK // NW
XTRA = NCHUNK % NW
RPS = N // NS
NBUF = 13

ROWB = 2000


@functools.cache
def _sc_mesh():
    return plsc.VectorSubcoreMesh(
        core_axis_name="c", subcore_axis_name="s", num_cores=NC, num_subcores=NS
    )


def _stage_idx(src_hbm, idx_ref, start, tile):
    @pl.when(tile < XTRA)
    def _full():
        pltpu.sync_copy(src_hbm.at[pl.ds(start, BASE + 1)], idx_ref)

    @pl.when(tile >= XTRA)
    def _base():
        pltpu.sync_copy(
            src_hbm.at[pl.ds(start, BASE)], idx_ref.at[pl.ds(0, BASE)]
        )


def _edge_views(ei_hbm):
    return ei_hbm.at[0], ei_hbm.at[1]


@functools.cache
def _sc_deg():
    @functools.partial(
        pl.kernel,
        out_type=jax.ShapeDtypeStruct((NC, N), jnp.float32),
        mesh=_sc_mesh(),
        scratch_types=[
            pltpu.VMEM((BASE + 1, CW), jnp.int32),
            pltpu.VMEM((CW,), jnp.float32),
            pltpu.VMEM_SHARED((N,), jnp.float32),
            pltpu.SemaphoreType.DMA,
        ],
        compiler_params=pltpu.CompilerParams(use_tc_tiling_on_sc=False),
    )
    def deg_kernel(ei_hbm, ones_hbm, zeros_hbm, out_hbm, idx_d, ones_v, acc_sh, sem):
        c = lax.axis_index("c")
        s = lax.axis_index("s")
        tile = c * NS + s
        _, dst_hbm = _edge_views(ei_hbm)

        @pl.when(s == 0)
        def _zero():
            pltpu.sync_copy(zeros_hbm, acc_sh)

        start = BASE * tile + jnp.minimum(tile, XTRA)
        _stage_idx(dst_hbm, idx_d, start, tile)
        pltpu.sync_copy(ones_hbm, ones_v)
        plsc.subcore_barrier()

        W = 8
        for b in range(W):
            pltpu.async_copy(ones_v, acc_sh.at[idx_d.at[b]], sem, add=True)

        @pl.loop(W, BASE)
        def _step(j):
            pltpu.make_async_copy(ones_v, acc_sh.at[pl.ds(0, CW)], sem).wait()
            pltpu.async_copy(ones_v, acc_sh.at[idx_d.at[j]], sem, add=True)

        @pl.when(tile < XTRA)
        def _extra():
            pltpu.make_async_copy(ones_v, acc_sh.at[pl.ds(0, CW)], sem).wait()
            pltpu.async_copy(ones_v, acc_sh.at[idx_d.at[BASE]], sem, add=True)

        for b in range(W):
            pltpu.make_async_copy(ones_v, acc_sh.at[pl.ds(0, CW)], sem).wait()
        plsc.subcore_barrier()

        @pl.when(s == 0)
        def _writeback():
            pltpu.sync_copy(acc_sh, out_hbm.at[c])

    return deg_kernel


@functools.cache
def _sc_agg(width, from_spmem):
    @functools.partial(
        pl.kernel,
        out_type=jax.ShapeDtypeStruct((NC, N, width), jnp.float32),
        mesh=_sc_mesh(),
        scratch_types=[
            pltpu.VMEM((BASE + 1, CW), jnp.int32),
            pltpu.VMEM((BASE + 1, CW), jnp.int32),
            pltpu.VMEM((NBUF, CW, width), jnp.float32),
            pltpu.VMEM_SHARED((N, width), jnp.float32),
            pltpu.VMEM_SHARED((N, width), jnp.float32),
            pltpu.SemaphoreType.DMA((NBUF,)),
            pltpu.SemaphoreType.DMA((NBUF,)),
        ],
        compiler_params=pltpu.CompilerParams(use_tc_tiling_on_sc=False),
    )
    def agg_kernel(g_hbm_flat, ei_hbm, zrows_hbm, out_hbm,
                   idx_s, idx_d, rows, acc_sh, g_sh, gsem, ssem):
        c = lax.axis_index("c")
        s = lax.axis_index("s")
        tile = c * NS + s
        src_hbm, dst_hbm = _edge_views(ei_hbm)
        pltpu.sync_copy(zrows_hbm, acc_sh.at[pl.ds(s * RPS, RPS)])
        if from_spmem:
            pltpu.sync_copy(
                g_hbm_flat.at[pl.ds(s * RPS, RPS)], g_sh.at[pl.ds(s * RPS, RPS)]
            )
            g_hbm = g_sh
        else:
            g_hbm = g_hbm_flat
        start = BASE * tile + jnp.minimum(tile, XTRA)
        _stage_idx(src_hbm, idx_s, start, tile)
        _stage_idx(dst_hbm, idx_d, start, tile)
        plsc.subcore_barrier()

        for b in range(NBUF):
            pltpu.async_copy(g_hbm.at[idx_s.at[b]], rows.at[b], gsem.at[b])

        @pl.loop(0, BASE, step=NBUF)
        def _group(j0):
            for b in range(NBUF):
                pltpu.make_async_copy(
                    g_hbm.at[idx_s.at[j0 + b]], rows.at[b], gsem.at[b]
                ).wait()
                pltpu.async_copy(
                    rows.at[b], acc_sh.at[idx_d.at[j0 + b]], ssem.at[b], add=True
                )
            for b in range(NBUF):
                jn = j0 + NBUF + b
                pltpu.make_async_copy(
                    rows.at[b], acc_sh.at[pl.ds(0, CW)], ssem.at[b]
                ).wait()

                @pl.when(jn < BASE + jnp.where(tile < XTRA, 1, 0))
                def _prefetch():
                    pltpu.async_copy(g_hbm.at[idx_s.at[jn]], rows.at[b], gsem.at[b])

        bt = BASE % NBUF

        @pl.when(tile < XTRA)
        def _tail():
            pltpu.make_async_copy(
                g_hbm.at[idx_s.at[BASE]], rows.at[bt], gsem.at[bt]
            ).wait()
            pltpu.async_copy(
                rows.at[bt], acc_sh.at[idx_d.at[BASE]], ssem.at[bt], add=True
            )
            pltpu.make_async_copy(
                rows.at[bt], acc_sh.at[pl.ds(0, CW)], ssem.at[bt]
            ).wait()

        plsc.subcore_barrier()
        pltpu.sync_copy(
            acc_sh.at[pl.ds(s * RPS, RPS)], out_hbm.at[c, pl.ds(s * RPS, RPS)]
        )

    return agg_kernel


PK = N * D_H // 128


def _pack(v, groups, w):
    rows = v.shape[0] // groups
    v3 = v.reshape(rows, groups, w)
    return jnp.concatenate([v3[:, i, :] for i in range(groups)], axis=1)


def _unpack(vp, groups, w):
    rows = vp.shape[0]
    parts = [vp[:, i * w:(i + 1) * w] for i in range(groups)]
    return jnp.stack(parts, axis=1).reshape(rows * groups, w)


def _k2_body(degp_ref, x_ref, w1_ref, g1_ref, dinv_ref, dinv8_ref):
    deg = degp_ref[0:1] + degp_ref[1:2] + 1.0
    dinv_col = jnp.transpose(lax.rsqrt(deg))
    dinv = jnp.broadcast_to(dinv_col, (N, D_H))
    h = jnp.dot(x_ref[...], w1_ref[...], preferred_element_type=jnp.float32)
    g1_ref[...] = _pack(h * dinv, 8, D_H)
    dinv_ref[...] = _pack(dinv, 8, D_H)
    dinv8_ref[...] = _pack(jnp.broadcast_to(dinv_col, (N, D2)), 8, D2)


def _tc_k2(degp, x, w1):
    return pl.pallas_call(
        _k2_body,
        out_shape=[
            jax.ShapeDtypeStruct((PK, 128), jnp.float32),
            jax.ShapeDtypeStruct((PK, 128), jnp.float32),
            jax.ShapeDtypeStruct((PK, 64), jnp.float32),
        ],
        compiler_params=pltpu.CompilerParams(vmem_limit_bytes=100 * 1024 * 1024),
    )(degp, x, w1)


def _k4_body(accp_ref, g1_ref, dinv_ref, b1t_ref, bd_ref, dinv8_ref, g2_ref):
    acc = accp_ref[0] + accp_ref[1] + g1_ref[...]
    h1 = jnp.maximum(acc * dinv_ref[...] + b1t_ref[...], 0.0)
    g2 = jnp.dot(h1, bd_ref[...], preferred_element_type=jnp.float32)
    g2_ref[...] = g2 * dinv8_ref[...]


def _tc_k4(accp, g1, dinv, b1t, bd, dinv8):
    return pl.pallas_call(
        _k4_body,
        out_shape=jax.ShapeDtypeStruct((PK, 64), jnp.float32),
        compiler_params=pltpu.CompilerParams(vmem_limit_bytes=100 * 1024 * 1024),
    )(accp, g1, dinv, b1t, bd, dinv8)


def _k6_body(accp_ref, g2_ref, dinv8_ref, b2t_ref, segsum_ref, out_ref):
    op = (accp_ref[0] + accp_ref[1] + g2_ref[...]) * dinv8_ref[...] + b2t_ref[...]
    lane = lax.broadcasted_iota(jnp.int32, (PK, 64), 1) % D2
    op = jnp.where(lane < D_OUT, op, -1e30)
    v = op
    for sh in (1, 2, 4):
        r = pltpu.roll(v, 64 - sh, 1)
        v = jnp.maximum(v, jnp.where(lane < D2 - sh, r, -1e30))
    m = v
    for sh in (1, 2, 4):
        r = pltpu.roll(m, sh, 1)
        m = jnp.maximum(m, jnp.where(lane >= sh, r, -1e30))
    ex = jnp.exp(op - m)
    ssum = jnp.dot(ex, segsum_ref[...], preferred_element_type=jnp.float32,
                   precision=lax.Precision.HIGHEST)
    out_ref[...] = op - m - jnp.log(ssum)


def _tc_k6(accp, g2, dinv8, b2t, segsum):
    return pl.pallas_call(
        _k6_body,
        out_shape=jax.ShapeDtypeStruct((PK, 64), jnp.float32),
        compiler_params=pltpu.CompilerParams(vmem_limit_bytes=100 * 1024 * 1024),
    )(accp, g2, dinv8, b2t, segsum)


def kernel(x, edge_index, W1, b1, W2, b2):
    zrows = jnp.zeros((RPS, D_H), jnp.float32)
    ones = jnp.ones((CW,), jnp.float32)
    zeros_n = jnp.zeros((N,), jnp.float32)

    ei3 = edge_index.reshape(2, NCHUNK, CW)
    degp = _sc_deg()(ei3, ones, zeros_n)
    g1p, dinvp, dinv8p = _tc_k2(degp, x, W1)
    accp1 = _sc_agg(D_H, False)(g1p.reshape(N, D_H), ei3, zrows)

    w2p = jnp.zeros((D_H, D2), jnp.float32).at[:, :D_OUT].set(W2)
    bd = jnp.kron(jnp.eye(8, dtype=jnp.float32), w2p)
    b1t = jnp.tile(b1, 8).reshape(1, 128)
    g2p = _tc_k4(accp1.reshape(NC, PK, 128), g1p, dinvp, b1t, bd, dinv8p)
    accp2 = _sc_agg(D2, True)(g2p.reshape(N, D2), ei3,
                        jnp.zeros((RPS, D2), jnp.float32))

    b2p8 = jnp.zeros((D2,), jnp.float32).at[:D_OUT].set(b2)
    b2t = jnp.tile(b2p8, 8).reshape(1, 64)
    segsum = jnp.kron(jnp.eye(8, dtype=jnp.float32), jnp.ones((D2, D2), jnp.float32))
    outp = _tc_k6(accp2.reshape(NC, PK, 64), g2p, dinv8p, b2t, segsum)
    return outp.reshape(N, D2)[:, :D_OUT]
